# baseline (speedup 1.0000x reference)
import functools

import jax
import jax.numpy as jnp
from jax import lax
from jax.experimental import pallas as pl
from jax.experimental.pallas import tpu as pltpu
from jax.experimental.pallas import tpu_sc as plsc

NF = 10000
NL = 10000
PLEN = 8
MP = 16
D = 64
ITERS = 8

NC, NS, LANES = 2, 16, 16
NW = NC * NS

_mesh = lambda: plsc.VectorSubcoreMesh(
    core_axis_name="c", subcore_axis_name="s", num_cores=NC, num_subcores=NS)


def _wid():
    return lax.axis_index("s") * NC + lax.axis_index("c")


_LPT = 320
_LBASE_MAX = NL - _LPT
_CPT = 2560
_CBASE_MAX = NF * PLEN - _CPT


def _sc_scalar_body(tr_hbm, cap_hbm, fi_hbm, l2pt_hbm, load_hbm, capg_hbm,
                    tr_v, cap_v, fi_v, l2pt_v, load_v, capg_v, sem):
    w = _wid()
    pltpu.sync_copy(tr_hbm, tr_v)
    pltpu.sync_copy(cap_hbm, cap_v)
    lbase = jnp.minimum(w * _LPT, _LBASE_MAX)
    cbase = jnp.minimum(w * _CPT, _CBASE_MAX)
    pltpu.sync_copy(fi_hbm.at[pl.ds(lbase * MP, _LPT * MP)], fi_v)
    pltpu.sync_copy(l2pt_hbm.at[pl.ds(cbase, _CPT)], l2pt_v)
    lanes = lax.iota(jnp.int32, LANES)

    def load_group(g, _):
        acc = jnp.zeros((LANES,), jnp.float32)
        for p in range(MP):
            fi = plsc.load_gather(fi_v, [lanes * MP + (g * LANES * MP + p)])
            acc = acc + plsc.load_gather(tr_v, [fi])
        capl = plsc.load_gather(cap_v, [lbase + g * LANES + lanes])
        load_v[pl.ds(g * LANES, LANES)] = acc / (capl * 1e9)
        return _

    lax.fori_loop(0, _LPT // LANES, load_group, 0)

    def capg_group(i, _):
        idxs = l2pt_v[pl.ds(i * LANES, LANES)]
        capg_v[pl.ds(i * LANES, LANES)] = plsc.load_gather(cap_v, [idxs])
        return _

    lax.fori_loop(0, _CPT // LANES, capg_group, 0)
    pltpu.sync_copy(load_v, load_hbm.at[pl.ds(lbase, _LPT)])
    pltpu.sync_copy(capg_v, capg_hbm.at[pl.ds(cbase, _CPT)])


def _sc_scalars(tr_flat, cap_flat, fi_flat, l2pt_flat):
    k = pl.kernel(
        _sc_scalar_body,
        out_type=(jax.ShapeDtypeStruct((NL,), jnp.float32),
                  jax.ShapeDtypeStruct((NF * PLEN,), jnp.float32)),
        mesh=_mesh(),
        compiler_params=pltpu.CompilerParams(needs_layout_passes=False),
        scratch_types=[
            pltpu.VMEM((NF,), jnp.float32),
            pltpu.VMEM((NL,), jnp.float32),
            pltpu.VMEM((_LPT * MP,), jnp.int32),
            pltpu.VMEM((_CPT,), jnp.int32),
            pltpu.VMEM((_LPT,), jnp.float32),
            pltpu.VMEM((_CPT,), jnp.float32),
            pltpu.SemaphoreType.DMA,
        ],
    )
    return k(tr_flat, cap_flat, fi_flat, l2pt_flat)


_CHUNK = 128


_NBUF = 4


def _sc_row_gather_body(n_rows, n_chunks, table_hbm, idx_hbm, out_hbm,
                        *scratch):
    idx_bufs = scratch[:_NBUF]
    row_bufs = scratch[_NBUF:2 * _NBUF]
    gsems = scratch[2 * _NBUF:3 * _NBUF]
    osems = scratch[3 * _NBUF:4 * _NBUF]
    w = _wid()
    per_w = n_chunks * _CHUNK
    base = jnp.minimum(w * per_w, n_rows - per_w)

    def issue_gather(c):
        b = c % _NBUF
        pltpu.sync_copy(idx_hbm.at[pl.ds(base + c * _CHUNK, _CHUNK)],
                        idx_bufs[b])
        pltpu.async_copy(table_hbm.at[idx_bufs[b]], row_bufs[b], gsems[b])

    def wait_writeback(c):
        b = c % _NBUF
        pltpu.make_async_copy(row_bufs[b],
                              out_hbm.at[pl.ds(base, _CHUNK)], osems[b]).wait()

    for j in range(min(_NBUF - 1, n_chunks)):
        issue_gather(j)
    for c in range(n_chunks):
        b = c % _NBUF
        pltpu.make_async_copy(table_hbm.at[idx_bufs[b]], row_bufs[b],
                              gsems[b]).wait()
        pltpu.async_copy(row_bufs[b],
                         out_hbm.at[pl.ds(base + c * _CHUNK, _CHUNK)],
                         osems[b])
        f = c + _NBUF - 1
        if f < n_chunks:
            if c >= 1:
                wait_writeback(c - 1)
            issue_gather(f)
    for c in range(max(0, n_chunks - _NBUF), n_chunks):
        wait_writeback(c)


def _sc_row_gather(table, idx_flat, n_rows):
    n_chunks = -(-n_rows // (NW * _CHUNK))
    body = functools.partial(_sc_row_gather_body, n_rows, n_chunks)
    k = pl.kernel(
        body,
        out_type=jax.ShapeDtypeStruct((n_rows, D), table.dtype),
        mesh=_mesh(),
        compiler_params=pltpu.CompilerParams(needs_layout_passes=False,
                                             use_tc_tiling_on_sc=False),
        scratch_types=(
            [pltpu.VMEM((_CHUNK,), jnp.int32)] * _NBUF
            + [pltpu.VMEM((_CHUNK, D), table.dtype)] * _NBUF
            + [pltpu.SemaphoreType.DMA] * (2 * _NBUF)
        ),
    )
    return k(table, idx_flat)


_R = 2000
_GRID = NF // _R


def _selu(x):
    alpha = 1.6732632423543772
    scale = 1.0507009873554805
    return scale * jnp.where(x > 0, x, alpha * (jnp.exp(x) - 1.0))


def _softplus(x):
    return jnp.maximum(x, 0.0) + jnp.log1p(jnp.exp(-jnp.abs(x)))


def _dot(a, b):
    return jax.lax.dot_general(a, b, (((1,), (0,)), ((), ())),
                               preferred_element_type=jnp.float32)


def _full(shape):
    return pl.BlockSpec(shape, lambda i: tuple(0 for _ in shape))


def _embed_body(pf_ref, lf_ref, pw1_ref, pb1_ref, pw2_ref, pb2_ref,
                lw1_ref, lb1_ref, lw2_ref, lb2_ref, ps_ref, ls_ref,
                lsb_ref):
    pf = pf_ref[...]
    ps_ref[...] = _selu(_dot(_selu(_dot(pf, pw1_ref[...]) + pb1_ref[...]),
                             pw2_ref[...]) + pb2_ref[...])
    lf = lf_ref[...]
    ls = _selu(_dot(_selu(_dot(lf, lw1_ref[...]) + lb1_ref[...]),
                    lw2_ref[...]) + lb2_ref[...])
    ls_ref[...] = ls
    lsb_ref[...] = ls.astype(jnp.bfloat16)


def _embed(pf, lf, pw1, pb1, pw2, pb2, lw1, lb1, lw2, lb2):
    return pl.pallas_call(
        _embed_body,
        grid=(_GRID,),
        in_specs=[
            pl.BlockSpec((_R, 5), lambda i: (i, 0)),
            pl.BlockSpec((_R, 2), lambda i: (i, 0)),
            _full((5, D)), _full((1, D)), _full((D, D)), _full((1, D)),
            _full((2, D)), _full((1, D)), _full((D, D)), _full((1, D)),
        ],
        out_specs=(pl.BlockSpec((_R, D), lambda i: (i, 0)),
                   pl.BlockSpec((_R, D), lambda i: (i, 0)),
                   pl.BlockSpec((_R, D), lambda i: (i, 0))),
        out_shape=(jax.ShapeDtypeStruct((NF, D), jnp.float32),
                   jax.ShapeDtypeStruct((NL, D), jnp.float32),
                   jax.ShapeDtypeStruct((NL, D), jnp.bfloat16)),
    )(pf, lf, pw1, pb1, pw2, pb2, lw1, lb1, lw2, lb2)


def _gru_step(x_gz, x_gr, x_gh, h, whz, whr, whh):
    z = jax.nn.sigmoid(x_gz + _dot(h, whz))
    r = jax.nn.sigmoid(x_gr + _dot(h, whr))
    cand = jnp.tanh(x_gh + r * _dot(h, whh))
    return z * h + (1.0 - z) * cand


def _pgru_body(xs_ref, h0_ref, wxz_ref, wxr_ref, wxh_ref,
               whz_ref, whr_ref, whh_ref, bz_ref, br_ref, bh_ref,
               out_ref, ht_ref):
    xs2 = xs_ref[...].reshape(PLEN * _R, D)
    gz = _dot(xs2, wxz_ref[...]) + bz_ref[...]
    gr = _dot(xs2, wxr_ref[...]) + br_ref[...]
    gh = _dot(xs2, wxh_ref[...]) + bh_ref[...]
    whz, whr, whh = whz_ref[...], whr_ref[...], whh_ref[...]
    h = h0_ref[...]
    out_ref[0] = h.astype(jnp.bfloat16)
    for t in range(PLEN):
        lo, hi = t * _R, (t + 1) * _R
        h = _gru_step(gz[lo:hi], gr[lo:hi], gh[lo:hi], h, whz, whr, whh)
        out_ref[t + 1] = h.astype(jnp.bfloat16)
    ht_ref[...] = h


def _pgru(xs, h0, wxz, wxr, wxh, whz, whr, whh, bz, br, bh):
    return pl.pallas_call(
        _pgru_body,
        grid=(_GRID,),
        in_specs=[
            pl.BlockSpec((PLEN, _R, D), lambda i: (0, i, 0)),
            pl.BlockSpec((_R, D), lambda i: (i, 0)),
            _full((D, D)), _full((D, D)), _full((D, D)),
            _full((D, D)), _full((D, D)), _full((D, D)),
            _full((1, D)), _full((1, D)), _full((1, D)),
        ],
        out_specs=(pl.BlockSpec((PLEN + 1, _R, D), lambda i: (0, i, 0)),
                   pl.BlockSpec((_R, D), lambda i: (i, 0))),
        out_shape=(jax.ShapeDtypeStruct((PLEN + 1, NF, D), jnp.bfloat16),
                   jax.ShapeDtypeStruct((NF, D), jnp.float32)),
    )(xs, h0, wxz, wxr, wxh, whz, whr, whh, bz, br, bh)


_RA = 1000
_GRIDA = NL // _RA


def _attn_body(pg_ref, ls_ref, aw_ref, ab_ref, wxz_ref, wxr_ref, wxh_ref,
               whz_ref, whr_ref, whh_ref, bz_ref, br_ref, bh_ref,
               out_ref, outb_ref):
    aw, ab = aw_ref[...], ab_ref[...]
    pgall = pg_ref[...].reshape(MP * _RA, D)
    coef = _dot(pgall, aw) + ab
    m = jnp.max(coef, axis=-1, keepdims=True)
    e = jnp.exp(coef - m)
    sm = e / jnp.sum(e, axis=-1, keepdims=True)
    acc = jnp.sum((sm * pgall.astype(jnp.float32)).reshape(MP, _RA, D),
                  axis=0)
    gz = _dot(acc, wxz_ref[...]) + bz_ref[...]
    gr = _dot(acc, wxr_ref[...]) + br_ref[...]
    gh = _dot(acc, wxh_ref[...]) + bh_ref[...]
    h = ls_ref[...]
    new_ls = _gru_step(gz, gr, gh, h,
                       whz_ref[...], whr_ref[...], whh_ref[...])
    out_ref[...] = new_ls
    outb_ref[...] = new_ls.astype(jnp.bfloat16)


def _attn(pg, ls, aw, ab, wxz, wxr, wxh, whz, whr, whh, bz, br, bh):
    return pl.pallas_call(
        _attn_body,
        grid=(_GRIDA,),
        in_specs=[
            pl.BlockSpec((MP, _RA, D), lambda i: (0, i, 0)),
            pl.BlockSpec((_RA, D), lambda i: (i, 0)),
            _full((D, D)), _full((1, D)),
            _full((D, D)), _full((D, D)), _full((D, D)),
            _full((D, D)), _full((D, D)), _full((D, D)),
            _full((1, D)), _full((1, D)), _full((1, D)),
        ],
        out_specs=(pl.BlockSpec((_RA, D), lambda i: (i, 0)),
                   pl.BlockSpec((_RA, D), lambda i: (i, 0))),
        out_shape=(jax.ShapeDtypeStruct((NL, D), jnp.float32),
                   jax.ShapeDtypeStruct((NL, D), jnp.bfloat16)),
    )(pg, ls, aw, ab, wxz, wxr, wxh, whz, whr, whh, bz, br, bh)


def _readout_body(pss_ref, capg_ref, w1_ref, b1_ref, w2_ref, b2_ref,
                  w3_ref, b3_ref, out_ref):
    w1, b1 = w1_ref[...], b1_ref[...]
    w2, b2 = w2_ref[...], b2_ref[...]
    w3, b3 = w3_ref[...], b3_ref[...]
    qd = jnp.zeros((_R, 1), jnp.float32)
    for t in range(PLEN):
        h1 = _selu(_dot(pss_ref[t + 1], w1) + b1)
        h2 = _selu(_dot(h1, w2) + b2)
        occ = _softplus(_dot(h2, w3) + b3)
        qd = qd + occ / capg_ref[t]
    out_ref[...] = qd


def _readout(pss, capg, w1, b1, w2, b2, w3, b3):
    return pl.pallas_call(
        _readout_body,
        grid=(_GRID,),
        in_specs=[
            pl.BlockSpec((PLEN + 1, _R, D), lambda i: (0, i, 0)),
            pl.BlockSpec((PLEN, _R, 1), lambda i: (0, i, 0)),
            _full((D, D // 2)), _full((1, D // 2)),
            _full((D // 2, D // 4)), _full((1, D // 4)),
            _full((D // 4, 1)), _full((1, 1)),
        ],
        out_specs=pl.BlockSpec((_R, 1), lambda i: (i, 0)),
        out_shape=jax.ShapeDtypeStruct((NF, 1), jnp.float32),
    )(pss, capg, w1, b1, w2, b2, w3, b3)


def kernel(flow_traffic, flow_packets, flow_packet_size, link_capacity,
           ibg, flow_on_rate, link_to_path, path_to_link, params):
    p = params
    f32 = jnp.float32

    fi = path_to_link[:, :, 0]
    si = path_to_link[:, :, 1]
    fi_flat = fi.reshape(-1)
    l2pt_flat = link_to_path.T.reshape(-1)
    pidx_flat = (si * NF + fi).T.reshape(-1)

    tr_flat = flow_traffic.reshape(-1)
    cap_flat = link_capacity.reshape(-1)

    load, capg_flat = _sc_scalars(tr_flat, cap_flat, fi_flat, l2pt_flat)
    capg = capg_flat.reshape(PLEN, NF, 1)

    pf = jnp.concatenate([flow_traffic, flow_packets, flow_packet_size,
                          ibg, flow_on_rate], axis=1)
    lf = jnp.concatenate([link_capacity, load[:, None]], axis=1)
    pw1 = 2.0 * p['pe_W1']
    pb1 = (p['pe_b1'] - p['pe_W1'].sum(axis=0))[None, :]
    lw1 = jnp.stack([2.0 * p['le_W1'][0], p['le_W1'][1]], axis=0)
    lb1 = (p['le_b1'] - p['le_W1'][0])[None, :]

    def split3(w):
        return w[:, :D], w[:, D:2 * D], w[:, 2 * D:]

    bf16 = jnp.bfloat16
    pu_wxz, pu_wxr, pu_wxh = (w.astype(bf16) for w in split3(p['pu_Wx']))
    pu_whz, pu_whr, pu_whh = split3(p['pu_Wh'])
    pu_bz = p['pu_b'][None, :D]
    pu_br = p['pu_b'][None, D:2 * D]
    pu_bh = p['pu_b'][None, 2 * D:]
    lu_wxz, lu_wxr, lu_wxh = split3(p['lu_Wx'])
    lu_whz, lu_whr, lu_whh = split3(p['lu_Wh'])
    lu_bz = p['lu_b'][None, :D]
    lu_br = p['lu_b'][None, D:2 * D]
    lu_bh = p['lu_b'][None, 2 * D:]

    path_state, link_state, ls_b = _embed(
        pf, lf, pw1, pb1, p['pe_W2'], p['pe_b2'][None, :],
        lw1, lb1, p['le_W2'], p['le_b2'][None, :])

    h0 = path_state
    pss = None
    for _ in range(ITERS):
        lg = _sc_row_gather(ls_b, l2pt_flat, NF * PLEN)
        xs = lg.reshape(PLEN, NF, D)
        pss, h0 = _pgru(xs, h0, pu_wxz, pu_wxr, pu_wxh,
                        pu_whz, pu_whr, pu_whh, pu_bz, pu_br, pu_bh)
        pg_rows = _sc_row_gather(pss.reshape((PLEN + 1) * NF, D),
                                 pidx_flat, NL * MP)
        pg = pg_rows.reshape(MP, NL, D)
        link_state, ls_b = _attn(
            pg, link_state, p['att_W'].astype(bf16), p['att_b'][None, :],
            lu_wxz, lu_wxr, lu_wxh, lu_whz, lu_whr, lu_whh,
            lu_bz, lu_br, lu_bh)

    return _readout(pss, capg, p['ro_W1'].astype(bf16), p['ro_b1'][None, :],
                    p['ro_W2'], p['ro_b2'][None, :],
                    p['ro_W3'], p['ro_b3'][None, :])


# speedup vs baseline: 1.2673x; 1.2673x over previous
import functools

import jax
import jax.numpy as jnp
from jax import lax
from jax.experimental import pallas as pl
from jax.experimental.pallas import tpu as pltpu
from jax.experimental.pallas import tpu_sc as plsc

NF = 10000
NL = 10000
PLEN = 8
MP = 16
D = 64
ITERS = 8

NC, NS, LANES = 2, 16, 16
NW = NC * NS

_mesh = lambda: plsc.VectorSubcoreMesh(
    core_axis_name="c", subcore_axis_name="s", num_cores=NC, num_subcores=NS)


def _wid():
    return lax.axis_index("s") * NC + lax.axis_index("c")


_LPT = 320
_LBASE_MAX = NL - _LPT
_CPT = 2560
_CBASE_MAX = NF * PLEN - _CPT


def _sc_scalar_body(tr_hbm, cap_hbm, fi_hbm, l2pt_hbm, load_hbm, capg_hbm,
                    tr_v, cap_v, fi_v, l2pt_v, load_v, capg_v, sem):
    w = _wid()
    pltpu.sync_copy(tr_hbm, tr_v)
    pltpu.sync_copy(cap_hbm, cap_v)
    lbase = jnp.minimum(w * _LPT, _LBASE_MAX)
    cbase = jnp.minimum(w * _CPT, _CBASE_MAX)
    pltpu.sync_copy(fi_hbm.at[pl.ds(lbase * MP, _LPT * MP)], fi_v)
    pltpu.sync_copy(l2pt_hbm.at[pl.ds(cbase, _CPT)], l2pt_v)
    lanes = lax.iota(jnp.int32, LANES)

    def load_group(g, _):
        acc = jnp.zeros((LANES,), jnp.float32)
        for p in range(MP):
            fi = plsc.load_gather(fi_v, [lanes * MP + (g * LANES * MP + p)])
            acc = acc + plsc.load_gather(tr_v, [fi])
        capl = plsc.load_gather(cap_v, [lbase + g * LANES + lanes])
        load_v[pl.ds(g * LANES, LANES)] = acc / (capl * 1e9)
        return _

    lax.fori_loop(0, _LPT // LANES, load_group, 0)

    def capg_group(i, _):
        idxs = l2pt_v[pl.ds(i * LANES, LANES)]
        capg_v[pl.ds(i * LANES, LANES)] = plsc.load_gather(cap_v, [idxs])
        return _

    lax.fori_loop(0, _CPT // LANES, capg_group, 0)
    pltpu.sync_copy(load_v, load_hbm.at[pl.ds(lbase, _LPT)])
    pltpu.sync_copy(capg_v, capg_hbm.at[pl.ds(cbase, _CPT)])


def _sc_scalars(tr_flat, cap_flat, fi_flat, l2pt_flat):
    k = pl.kernel(
        _sc_scalar_body,
        out_type=(jax.ShapeDtypeStruct((NL,), jnp.float32),
                  jax.ShapeDtypeStruct((NF * PLEN,), jnp.float32)),
        mesh=_mesh(),
        compiler_params=pltpu.CompilerParams(needs_layout_passes=False),
        scratch_types=[
            pltpu.VMEM((NF,), jnp.float32),
            pltpu.VMEM((NL,), jnp.float32),
            pltpu.VMEM((_LPT * MP,), jnp.int32),
            pltpu.VMEM((_CPT,), jnp.int32),
            pltpu.VMEM((_LPT,), jnp.float32),
            pltpu.VMEM((_CPT,), jnp.float32),
            pltpu.SemaphoreType.DMA,
        ],
    )
    return k(tr_flat, cap_flat, fi_flat, l2pt_flat)


_CHUNK = 128


_NBUF = 6


def _sc_row_gather_body(n_rows, n_chunks, table_hbm, idx_hbm, out_hbm,
                        *scratch):
    idx_bufs = scratch[:_NBUF]
    row_bufs = scratch[_NBUF:2 * _NBUF]
    gsems = scratch[2 * _NBUF:3 * _NBUF]
    osems = scratch[3 * _NBUF:4 * _NBUF]
    w = _wid()
    per_w = n_chunks * _CHUNK
    base = jnp.minimum(w * per_w, n_rows - per_w)

    def issue_gather(c):
        b = c % _NBUF
        pltpu.sync_copy(idx_hbm.at[pl.ds(base + c * _CHUNK, _CHUNK)],
                        idx_bufs[b])
        pltpu.async_copy(table_hbm.at[idx_bufs[b]], row_bufs[b], gsems[b])

    def wait_writeback(c):
        b = c % _NBUF
        pltpu.make_async_copy(row_bufs[b],
                              out_hbm.at[pl.ds(base, _CHUNK)], osems[b]).wait()

    for j in range(min(_NBUF - 1, n_chunks)):
        issue_gather(j)
    for c in range(n_chunks):
        b = c % _NBUF
        pltpu.make_async_copy(table_hbm.at[idx_bufs[b]], row_bufs[b],
                              gsems[b]).wait()
        pltpu.async_copy(row_bufs[b],
                         out_hbm.at[pl.ds(base + c * _CHUNK, _CHUNK)],
                         osems[b])
        f = c + _NBUF - 1
        if f < n_chunks:
            if c >= 1:
                wait_writeback(c - 1)
            issue_gather(f)
    for c in range(max(0, n_chunks - _NBUF), n_chunks):
        wait_writeback(c)


def _sc_row_gather(table, idx_flat, n_rows):
    n_chunks = -(-n_rows // (NW * _CHUNK))
    body = functools.partial(_sc_row_gather_body, n_rows, n_chunks)
    k = pl.kernel(
        body,
        out_type=jax.ShapeDtypeStruct((n_rows, D), table.dtype),
        mesh=_mesh(),
        compiler_params=pltpu.CompilerParams(needs_layout_passes=False,
                                             use_tc_tiling_on_sc=False),
        scratch_types=(
            [pltpu.VMEM((_CHUNK,), jnp.int32)] * _NBUF
            + [pltpu.VMEM((_CHUNK, D), table.dtype)] * _NBUF
            + [pltpu.SemaphoreType.DMA] * (2 * _NBUF)
        ),
    )
    return k(table, idx_flat)


_R = 2000
_GRID = NF // _R


def _selu(x):
    alpha = 1.6732632423543772
    scale = 1.0507009873554805
    return scale * jnp.where(x > 0, x, alpha * (jnp.exp(x) - 1.0))


def _softplus(x):
    return jnp.maximum(x, 0.0) + jnp.log1p(jnp.exp(-jnp.abs(x)))


def _dot(a, b):
    return jax.lax.dot_general(a, b, (((1,), (0,)), ((), ())),
                               preferred_element_type=jnp.float32)


def _full(shape):
    return pl.BlockSpec(shape, lambda i: tuple(0 for _ in shape))


def _embed_body(pf_ref, lf_ref, pw1_ref, pb1_ref, pw2_ref, pb2_ref,
                lw1_ref, lb1_ref, lw2_ref, lb2_ref, ps_ref, ls_ref):
    pf = pf_ref[...]
    ps_ref[...] = _selu(_dot(_selu(_dot(pf, pw1_ref[...]) + pb1_ref[...]),
                             pw2_ref[...]) + pb2_ref[...])
    lf = lf_ref[...]
    ls_ref[...] = _selu(_dot(_selu(_dot(lf, lw1_ref[...]) + lb1_ref[...]),
                             lw2_ref[...]) + lb2_ref[...])


def _embed(pf, lf, pw1, pb1, pw2, pb2, lw1, lb1, lw2, lb2):
    return pl.pallas_call(
        _embed_body,
        grid=(_GRID,),
        in_specs=[
            pl.BlockSpec((_R, 5), lambda i: (i, 0)),
            pl.BlockSpec((_R, 2), lambda i: (i, 0)),
            _full((5, D)), _full((1, D)), _full((D, D)), _full((1, D)),
            _full((2, D)), _full((1, D)), _full((D, D)), _full((1, D)),
        ],
        out_specs=(pl.BlockSpec((_R, D), lambda i: (i, 0)),
                   pl.BlockSpec((_R, D), lambda i: (i, 0))),
        out_shape=(jax.ShapeDtypeStruct((NF, D), jnp.float32),
                   jax.ShapeDtypeStruct((NL, D), jnp.float32)),
    )(pf, lf, pw1, pb1, pw2, pb2, lw1, lb1, lw2, lb2)


def _gru_step(x_gz, x_gr, x_gh, h, whz, whr, whh):
    z = jax.nn.sigmoid(x_gz + _dot(h, whz))
    r = jax.nn.sigmoid(x_gr + _dot(h, whr))
    cand = jnp.tanh(x_gh + r * _dot(h, whh))
    return z * h + (1.0 - z) * cand


def _pgru_body(xs_ref, h0_ref, wxz_ref, wxr_ref, wxh_ref,
               whz_ref, whr_ref, whh_ref, bz_ref, br_ref, bh_ref,
               out_ref):
    xs2 = xs_ref[...].reshape(PLEN * _R, D)
    gz = _dot(xs2, wxz_ref[...]) + bz_ref[...]
    gr = _dot(xs2, wxr_ref[...]) + br_ref[...]
    gh = _dot(xs2, wxh_ref[...]) + bh_ref[...]
    whz, whr, whh = whz_ref[...], whr_ref[...], whh_ref[...]
    h = h0_ref[...]
    out_ref[0] = h
    for t in range(PLEN):
        lo, hi = t * _R, (t + 1) * _R
        h = _gru_step(gz[lo:hi], gr[lo:hi], gh[lo:hi], h, whz, whr, whh)
        out_ref[t + 1] = h


def _pgru(xs, h0, wxz, wxr, wxh, whz, whr, whh, bz, br, bh):
    return pl.pallas_call(
        _pgru_body,
        grid=(_GRID,),
        in_specs=[
            pl.BlockSpec((PLEN, _R, D), lambda i: (0, i, 0)),
            pl.BlockSpec((_R, D), lambda i: (i, 0)),
            _full((D, D)), _full((D, D)), _full((D, D)),
            _full((D, D)), _full((D, D)), _full((D, D)),
            _full((1, D)), _full((1, D)), _full((1, D)),
        ],
        out_specs=pl.BlockSpec((PLEN + 1, _R, D), lambda i: (0, i, 0)),
        out_shape=jax.ShapeDtypeStruct((PLEN + 1, NF, D), jnp.float32),
    )(xs, h0, wxz, wxr, wxh, whz, whr, whh, bz, br, bh)


_RA = 1000
_GRIDA = NL // _RA


def _attn_body(pg_ref, ls_ref, aw_ref, ab_ref, wxz_ref, wxr_ref, wxh_ref,
               whz_ref, whr_ref, whh_ref, bz_ref, br_ref, bh_ref,
               out_ref):
    aw, ab = aw_ref[...], ab_ref[...]
    pgall = pg_ref[...].reshape(MP * _RA, D)
    coef = _dot(pgall, aw) + ab
    m = jnp.max(coef, axis=-1, keepdims=True)
    e = jnp.exp(coef - m)
    sm = e / jnp.sum(e, axis=-1, keepdims=True)
    acc = jnp.sum((sm * pgall).reshape(MP, _RA, D), axis=0)
    gz = _dot(acc, wxz_ref[...]) + bz_ref[...]
    gr = _dot(acc, wxr_ref[...]) + br_ref[...]
    gh = _dot(acc, wxh_ref[...]) + bh_ref[...]
    h = ls_ref[...]
    out_ref[...] = _gru_step(gz, gr, gh, h,
                             whz_ref[...], whr_ref[...], whh_ref[...])


def _attn(pg, ls, aw, ab, wxz, wxr, wxh, whz, whr, whh, bz, br, bh):
    return pl.pallas_call(
        _attn_body,
        grid=(_GRIDA,),
        in_specs=[
            pl.BlockSpec((MP, _RA, D), lambda i: (0, i, 0)),
            pl.BlockSpec((_RA, D), lambda i: (i, 0)),
            _full((D, D)), _full((1, D)),
            _full((D, D)), _full((D, D)), _full((D, D)),
            _full((D, D)), _full((D, D)), _full((D, D)),
            _full((1, D)), _full((1, D)), _full((1, D)),
        ],
        out_specs=pl.BlockSpec((_RA, D), lambda i: (i, 0)),
        out_shape=jax.ShapeDtypeStruct((NL, D), jnp.float32),
    )(pg, ls, aw, ab, wxz, wxr, wxh, whz, whr, whh, bz, br, bh)


def _readout_body(pss_ref, capg_ref, w1_ref, b1_ref, w2_ref, b2_ref,
                  w3_ref, b3_ref, out_ref):
    w1, b1 = w1_ref[...], b1_ref[...]
    w2, b2 = w2_ref[...], b2_ref[...]
    w3, b3 = w3_ref[...], b3_ref[...]
    qd = jnp.zeros((_R, 1), jnp.float32)
    for t in range(PLEN):
        h1 = _selu(_dot(pss_ref[t + 1], w1) + b1)
        h2 = _selu(_dot(h1, w2) + b2)
        occ = _softplus(_dot(h2, w3) + b3)
        qd = qd + occ / capg_ref[t]
    out_ref[...] = qd


def _readout(pss, capg, w1, b1, w2, b2, w3, b3):
    return pl.pallas_call(
        _readout_body,
        grid=(_GRID,),
        in_specs=[
            pl.BlockSpec((PLEN + 1, _R, D), lambda i: (0, i, 0)),
            pl.BlockSpec((PLEN, _R, 1), lambda i: (0, i, 0)),
            _full((D, D // 2)), _full((1, D // 2)),
            _full((D // 2, D // 4)), _full((1, D // 4)),
            _full((D // 4, 1)), _full((1, 1)),
        ],
        out_specs=pl.BlockSpec((_R, 1), lambda i: (i, 0)),
        out_shape=jax.ShapeDtypeStruct((NF, 1), jnp.float32),
    )(pss, capg, w1, b1, w2, b2, w3, b3)


def kernel(flow_traffic, flow_packets, flow_packet_size, link_capacity,
           ibg, flow_on_rate, link_to_path, path_to_link, params):
    p = params
    f32 = jnp.float32

    fi = path_to_link[:, :, 0]
    si = path_to_link[:, :, 1]
    fi_flat = fi.reshape(-1)
    l2pt_flat = link_to_path.T.reshape(-1)
    pidx_flat = (si * NF + fi).T.reshape(-1)

    tr_flat = flow_traffic.reshape(-1)
    cap_flat = link_capacity.reshape(-1)

    load, capg_flat = _sc_scalars(tr_flat, cap_flat, fi_flat, l2pt_flat)
    capg = capg_flat.reshape(PLEN, NF, 1)

    pf = jnp.concatenate([flow_traffic, flow_packets, flow_packet_size,
                          ibg, flow_on_rate], axis=1)
    lf = jnp.concatenate([link_capacity, load[:, None]], axis=1)
    pw1 = 2.0 * p['pe_W1']
    pb1 = (p['pe_b1'] - p['pe_W1'].sum(axis=0))[None, :]
    lw1 = jnp.stack([2.0 * p['le_W1'][0], p['le_W1'][1]], axis=0)
    lb1 = (p['le_b1'] - p['le_W1'][0])[None, :]

    def split3(w):
        return w[:, :D], w[:, D:2 * D], w[:, 2 * D:]

    pu_wxz, pu_wxr, pu_wxh = split3(p['pu_Wx'])
    pu_whz, pu_whr, pu_whh = split3(p['pu_Wh'])
    pu_bz = p['pu_b'][None, :D]
    pu_br = p['pu_b'][None, D:2 * D]
    pu_bh = p['pu_b'][None, 2 * D:]
    lu_wxz, lu_wxr, lu_wxh = split3(p['lu_Wx'])
    lu_whz, lu_whr, lu_whh = split3(p['lu_Wh'])
    lu_bz = p['lu_b'][None, :D]
    lu_br = p['lu_b'][None, D:2 * D]
    lu_bh = p['lu_b'][None, 2 * D:]

    path_state, link_state = _embed(
        pf, lf, pw1, pb1, p['pe_W2'], p['pe_b2'][None, :],
        lw1, lb1, p['le_W2'], p['le_b2'][None, :])

    h0 = path_state
    pss = None
    for _ in range(ITERS):
        lg = _sc_row_gather(link_state, l2pt_flat, NF * PLEN)
        xs = lg.reshape(PLEN, NF, D)
        pss = _pgru(xs, h0, pu_wxz, pu_wxr, pu_wxh,
                    pu_whz, pu_whr, pu_whh, pu_bz, pu_br, pu_bh)
        pg_rows = _sc_row_gather(pss.reshape((PLEN + 1) * NF, D),
                                 pidx_flat, NL * MP)
        pg = pg_rows.reshape(MP, NL, D)
        link_state = _attn(pg, link_state, p['att_W'], p['att_b'][None, :],
                           lu_wxz, lu_wxr, lu_wxh, lu_whz, lu_whr, lu_whh,
                           lu_bz, lu_br, lu_bh)
        h0 = pss[PLEN]

    return _readout(pss, capg, p['ro_W1'], p['ro_b1'][None, :],
                    p['ro_W2'], p['ro_b2'][None, :],
                    p['ro_W3'], p['ro_b3'][None, :])
